# baseline (speedup 1.0000x reference)
import functools

import jax
import jax.numpy as jnp
from jax import lax
from jax.experimental import pallas as pl
from jax.experimental.pallas import tpu as pltpu

BQ = 512
BK = 608
_MASKED = 1e10
_DEAD = 3e38
_IDX_BIG = 1 << 30


def _insert3(run, c_v, c_i):
    (v0, i0), (v1, i1), (v2, i2) = run
    out = []
    for v, idx in ((v0, i0), (v1, i1), (v2, i2)):
        lt = c_v < v
        out.append((jnp.minimum(c_v, v), jnp.where(lt, c_i, idx)))
        c_v = jnp.maximum(c_v, v)
        c_i = jnp.where(lt, idx, c_i)
    return tuple(out)


def _fused_kernel(astart_ref, nt_ref,
                  qpT_ref, xs_ref, bs_ref,
                  posS_ref, batchS_ref, x_ref,
                  W1_ref, b1_ref, W2_ref, b2_ref,
                  out_ref,
                  *, din, dskip):
    i = pl.program_id(0)
    a0 = astart_ref[i]
    ntile = nt_ref[i]
    n = x_ref.shape[0]

    bs = bs_ref[0, 0, :]
    qc = [qpT_ref[c, :] for c in range(3)]

    iota_s = lax.broadcasted_iota(jnp.int32, (BK, BQ), 0)
    iota_k = lax.broadcasted_iota(jnp.int32, (BQ, BK), 1)

    init = tuple((jnp.full((BQ,), _MASKED, jnp.float32),
                  jnp.full((BQ,), k, jnp.int32)) for k in range(3))

    def dist_body(t, run):
        start = a0 + t * BK
        base = pl.multiple_of(jnp.minimum(start, n - BK), 8)
        d2 = jnp.zeros((BK, BQ), jnp.float32)
        for c in range(3):
            pc = posS_ref[pl.ds(base, BK), c]
            diff = pc[:, None] - qc[c][None, :]
            d2 = d2 + diff * diff
        bt = batchS_ref[pl.ds(base, BK), 0]
        d2 = jnp.where(bt[:, None] != bs[None, :], _MASKED, d2)
        li = iota_s + base
        d2 = jnp.where(li < start, _DEAD, d2)
        for p in range(3):
            vm = jnp.min(d2, axis=0)
            im = jnp.min(jnp.where(d2 == vm[None, :], li, _IDX_BIG), axis=0)
            run = _insert3(run, vm, im)
            if p < 2:
                d2 = jnp.where(li == im[None, :], _DEAD, d2)
        return run

    run = lax.fori_loop(0, ntile, dist_body, init)

    ws = [1.0 / jnp.maximum(v, 1e-16) for v, _ in run]
    wsum = ws[0] + ws[1] + ws[2]
    wn = [w / wsum for w in ws]

    def gath_body(t, acc):
        start = a0 + t * BK
        base = pl.multiple_of(jnp.minimum(start, n - BK), 8)
        xt = x_ref[pl.ds(base, BK), :]
        gi = iota_k + base
        gi = jnp.where(gi >= start, gi, -1)
        a = jnp.zeros((BQ, BK), jnp.float32)
        for kk in (2, 1, 0):
            a = jnp.where(gi == run[kk][1][:, None], wn[kk][:, None], a)
        return acc + jnp.dot(a, xt, preferred_element_type=jnp.float32)

    y = lax.fori_loop(0, ntile, gath_body, jnp.zeros((BQ, din), jnp.float32))

    xs = xs_ref[...]
    h = jnp.dot(y, W1_ref[:din, :], preferred_element_type=jnp.float32)
    h = h + jnp.dot(xs, W1_ref[din:din + dskip, :],
                    preferred_element_type=jnp.float32)
    h = jnp.maximum(h + b1_ref[...], 0.0)
    out = jnp.dot(h, W2_ref[...], preferred_element_type=jnp.float32)
    out_ref[...] = out + b2_ref[...]


def kernel(x, pos, batch, x_skip, pos_skip, batch_skip, W1, b1, W2, b2):
    n, din = x.shape
    ns, dskip = x_skip.shape
    dh = W1.shape[1]
    dout = W2.shape[1]
    g = ns // BQ

    batch32 = batch.astype(jnp.int32)
    bs32 = batch_skip.astype(jnp.int32)

    b_lo = bs32[0::BQ]
    b_hi = bs32[BQ - 1::BQ]
    row_start = jnp.sum(batch32[None, :] < b_lo[:, None],
                        axis=1, dtype=jnp.int32)
    row_end = jnp.sum(batch32[None, :] <= b_hi[:, None],
                      axis=1, dtype=jnp.int32)
    astart = (row_start // 8) * 8
    nt = jnp.maximum(0, (row_end - astart + BK - 1) // BK)

    qpT = pos_skip.T
    batchS = batch32.reshape(n, 1)
    bs3 = bs32.reshape(g, 1, BQ)
    b1r = b1.reshape(1, dh)
    b2r = b2.reshape(1, dout)

    grid_spec = pltpu.PrefetchScalarGridSpec(
        num_scalar_prefetch=2,
        grid=(g,),
        in_specs=[
            pl.BlockSpec((3, BQ), lambda i, *_: (0, i)),
            pl.BlockSpec((BQ, dskip), lambda i, *_: (i, 0)),
            pl.BlockSpec((1, 1, BQ), lambda i, *_: (i, 0, 0)),
            pl.BlockSpec((n, 3), lambda i, *_: (0, 0)),
            pl.BlockSpec((n, 1), lambda i, *_: (0, 0)),
            pl.BlockSpec((n, din), lambda i, *_: (0, 0)),
            pl.BlockSpec((din + dskip, dh), lambda i, *_: (0, 0)),
            pl.BlockSpec((1, dh), lambda i, *_: (0, 0)),
            pl.BlockSpec((dh, dout), lambda i, *_: (0, 0)),
            pl.BlockSpec((1, dout), lambda i, *_: (0, 0)),
        ],
        out_specs=pl.BlockSpec((BQ, dout), lambda i, *_: (i, 0)),
    )

    out = pl.pallas_call(
        functools.partial(_fused_kernel, din=din, dskip=dskip),
        grid_spec=grid_spec,
        out_shape=jax.ShapeDtypeStruct((ns, dout), jnp.float32),
    )(astart, nt, qpT, x_skip, bs3, pos, batchS, x, W1, b1r, W2, b2r)

    return (out, pos_skip, batch_skip)


# speedup vs baseline: 1.0167x; 1.0167x over previous
import functools

import jax
import jax.numpy as jnp
from jax import lax
from jax.experimental import pallas as pl
from jax.experimental.pallas import tpu as pltpu

BQ = 512
BK = 576
_MASKED = 1e10
_DEAD = 3e38
_IDX_BIG = 1 << 30


def _insert3(run, c_v, c_i):
    (v0, i0), (v1, i1), (v2, i2) = run
    out = []
    for v, idx in ((v0, i0), (v1, i1), (v2, i2)):
        lt = c_v < v
        out.append((jnp.minimum(c_v, v), jnp.where(lt, c_i, idx)))
        c_v = jnp.maximum(c_v, v)
        c_i = jnp.where(lt, idx, c_i)
    return tuple(out)


def _fused_kernel(astart_ref, nt_ref,
                  qpT_ref, xs_ref, bs_ref,
                  posS_ref, batchS_ref, x_ref,
                  W1_ref, b1_ref, W2_ref, b2_ref,
                  out_ref,
                  *, din, dskip):
    i = pl.program_id(0)
    a0 = astart_ref[i]
    ntile = nt_ref[i]
    n = x_ref.shape[0]

    bs = bs_ref[0, 0, :]
    qc = [qpT_ref[c, :] for c in range(3)]

    iota_s = lax.broadcasted_iota(jnp.int32, (BK, BQ), 0)
    iota_k = lax.broadcasted_iota(jnp.int32, (BQ, BK), 1)

    init = tuple((jnp.full((BQ,), _MASKED, jnp.float32),
                  jnp.full((BQ,), k, jnp.int32)) for k in range(3))

    def dist_body(t, run):
        start = a0 + t * BK
        base = pl.multiple_of(jnp.minimum(start, n - BK), 8)
        d2 = jnp.zeros((BK, BQ), jnp.float32)
        for c in range(3):
            pc = posS_ref[pl.ds(base, BK), c]
            diff = pc[:, None] - qc[c][None, :]
            d2 = d2 + diff * diff
        bt = batchS_ref[pl.ds(base, BK), 0]
        d2 = jnp.where(bt[:, None] != bs[None, :], _MASKED, d2)
        li = iota_s + base
        d2 = jnp.where(li < start, _DEAD, d2)
        for p in range(3):
            vm = jnp.min(d2, axis=0)
            im = jnp.min(jnp.where(d2 == vm[None, :], li, _IDX_BIG), axis=0)
            run = _insert3(run, vm, im)
            if p < 2:
                d2 = jnp.where(li == im[None, :], _DEAD, d2)
        return run

    run = lax.fori_loop(0, ntile, dist_body, init)

    ws = [1.0 / jnp.maximum(v, 1e-16) for v, _ in run]
    wsum = ws[0] + ws[1] + ws[2]
    wn = [w / wsum for w in ws]

    def gath_body(t, acc):
        start = a0 + t * BK
        base = pl.multiple_of(jnp.minimum(start, n - BK), 8)
        xt = x_ref[pl.ds(base, BK), :]
        gi = iota_k + base
        gi = jnp.where(gi >= start, gi, -1)
        a = jnp.zeros((BQ, BK), jnp.float32)
        for kk in (2, 1, 0):
            a = jnp.where(gi == run[kk][1][:, None], wn[kk][:, None], a)
        return acc + jnp.dot(a, xt, preferred_element_type=jnp.float32)

    y = lax.fori_loop(0, ntile, gath_body, jnp.zeros((BQ, din), jnp.float32))

    xs = xs_ref[...]
    h = jnp.dot(y, W1_ref[:din, :], preferred_element_type=jnp.float32)
    h = h + jnp.dot(xs, W1_ref[din:din + dskip, :],
                    preferred_element_type=jnp.float32)
    h = jnp.maximum(h + b1_ref[...], 0.0)
    out = jnp.dot(h, W2_ref[...], preferred_element_type=jnp.float32)
    out_ref[...] = out + b2_ref[...]


def kernel(x, pos, batch, x_skip, pos_skip, batch_skip, W1, b1, W2, b2):
    n, din = x.shape
    ns, dskip = x_skip.shape
    dh = W1.shape[1]
    dout = W2.shape[1]
    g = ns // BQ

    batch32 = batch.astype(jnp.int32)
    bs32 = batch_skip.astype(jnp.int32)

    b_lo = bs32[0::BQ]
    b_hi = bs32[BQ - 1::BQ]
    row_start = jnp.sum(batch32[None, :] < b_lo[:, None],
                        axis=1, dtype=jnp.int32)
    row_end = jnp.sum(batch32[None, :] <= b_hi[:, None],
                      axis=1, dtype=jnp.int32)
    astart = (row_start // 8) * 8
    nt = jnp.maximum(0, (row_end - astart + BK - 1) // BK)

    qpT = pos_skip.T
    batchS = batch32.reshape(n, 1)
    bs3 = bs32.reshape(g, 1, BQ)
    b1r = b1.reshape(1, dh)
    b2r = b2.reshape(1, dout)

    grid_spec = pltpu.PrefetchScalarGridSpec(
        num_scalar_prefetch=2,
        grid=(g,),
        in_specs=[
            pl.BlockSpec((3, BQ), lambda i, *_: (0, i)),
            pl.BlockSpec((BQ, dskip), lambda i, *_: (i, 0)),
            pl.BlockSpec((1, 1, BQ), lambda i, *_: (i, 0, 0)),
            pl.BlockSpec((n, 3), lambda i, *_: (0, 0)),
            pl.BlockSpec((n, 1), lambda i, *_: (0, 0)),
            pl.BlockSpec((n, din), lambda i, *_: (0, 0)),
            pl.BlockSpec((din + dskip, dh), lambda i, *_: (0, 0)),
            pl.BlockSpec((1, dh), lambda i, *_: (0, 0)),
            pl.BlockSpec((dh, dout), lambda i, *_: (0, 0)),
            pl.BlockSpec((1, dout), lambda i, *_: (0, 0)),
        ],
        out_specs=pl.BlockSpec((BQ, dout), lambda i, *_: (i, 0)),
    )

    out = pl.pallas_call(
        functools.partial(_fused_kernel, din=din, dskip=dskip),
        grid_spec=grid_spec,
        out_shape=jax.ShapeDtypeStruct((ns, dout), jnp.float32),
    )(astart, nt, qpT, x_skip, bs3, pos, batchS, x, W1, b1r, W2, b2r)

    return (out, pos_skip, batch_skip)
